# baseline (speedup 1.0000x reference)
import functools

import jax
import jax.numpy as jnp
import numpy as np
from jax import lax
from jax.experimental import pallas as pl
from jax.experimental.pallas import tpu as pltpu
from jax.experimental.pallas import tpu_sc as plsc

N = 50000
E = 100000
R = 6
D = 128
DE = 16
H = 4
DH = D // H

NT = 256
N_PAD = 50176
E_PAD = 102400
NBLK_N = N_PAD // NT
NBLK_E = E_PAD // NT
NP = 9
CW = 16

NC, NS = 2, 16
CS = 128
GB = R * E_PAD
PER_W = GB // (NC * NS)
GCH = PER_W // CS
RPS = N_PAD // NS
EPT = E_PAD // NS
SCH = EPT // CS


def _gelu(z):
    return 0.5 * z * (1.0 + lax.erf(z * 0.7071067811865476))


def _layernorm(z, g, b, eps=1e-5):
    m = jnp.mean(z, axis=-1, keepdims=True)
    v = jnp.mean((z - m) ** 2, axis=-1, keepdims=True)
    return (z - m) / jnp.sqrt(v + eps) * g + b


def _proj_body(x_ref, w_ref, b_ref, o_ref):
    o_ref[0] = (
        jnp.dot(x_ref[...], w_ref[0], preferred_element_type=jnp.float32)
        + b_ref[0]
    )


def _tc_qall(x_pad, wq, bq):
    return pl.pallas_call(
        _proj_body,
        grid=(R, NBLK_N),
        in_specs=[
            pl.BlockSpec((NT, D), lambda r, i: (i, 0)),
            pl.BlockSpec((1, D, D), lambda r, i: (r, 0, 0)),
            pl.BlockSpec((1, 1, D), lambda r, i: (r, 0, 0)),
        ],
        out_specs=pl.BlockSpec((1, NT, D), lambda r, i: (r, i, 0)),
        out_shape=jax.ShapeDtypeStruct((R, N_PAD, D), jnp.float32),
    )(x_pad, wq, bq)


def _edge_body(xs_ref, ea_ref, qd_ref, wk1_ref, wk2_ref, bk_ref,
               wv1_ref, wv2_ref, bv_ref, o_ref):
    i = pl.program_id(1)
    xs = xs_ref[0]
    ea = ea_ref[0]
    qd = qd_ref[0]
    k = (jnp.dot(xs, wk1_ref[0], preferred_element_type=jnp.float32)
         + jnp.dot(ea, wk2_ref[0], preferred_element_type=jnp.float32)
         + bk_ref[0])
    v = (jnp.dot(xs, wv1_ref[0], preferred_element_type=jnp.float32)
         + jnp.dot(ea, wv2_ref[0], preferred_element_type=jnp.float32)
         + bv_ref[0])
    drow = lax.broadcasted_iota(jnp.int32, (D, H), 0) // DH
    hcol = lax.broadcasted_iota(jnp.int32, (D, H), 1)
    sel = (drow == hcol).astype(jnp.float32)
    s = jnp.dot(qd * k, sel, preferred_element_type=jnp.float32)
    rowid = i * NT + lax.broadcasted_iota(jnp.int32, (NT, 1), 0)
    e = jnp.where(rowid < E, jnp.exp(s), 0.0)
    selt = (jnp.transpose(hcol) == jnp.transpose(drow)).astype(jnp.float32)
    ev = v * jnp.dot(e, selt, preferred_element_type=jnp.float32)
    for p in range(NP - 1):
        o_ref[p, 0] = ev[:, p * CW:(p + 1) * CW]
    prow = lax.broadcasted_iota(jnp.int32, (H, CW), 0)
    pcol = lax.broadcasted_iota(jnp.int32, (H, CW), 1)
    pad = (prow == pcol).astype(jnp.float32)
    o_ref[NP - 1, 0] = jnp.dot(e, pad, preferred_element_type=jnp.float32)


def _tc_edges(xs, ea, qd, wk1, wk2, bk, wv1, wv2, bv):
    return pl.pallas_call(
        _edge_body,
        grid=(R, NBLK_E),
        in_specs=[
            pl.BlockSpec((1, NT, D), lambda r, i: (r, i, 0)),
            pl.BlockSpec((1, NT, DE), lambda r, i: (r, i, 0)),
            pl.BlockSpec((1, NT, D), lambda r, i: (r, i, 0)),
            pl.BlockSpec((1, D, D), lambda r, i: (r, 0, 0)),
            pl.BlockSpec((1, DE, D), lambda r, i: (r, 0, 0)),
            pl.BlockSpec((1, 1, D), lambda r, i: (r, 0, 0)),
            pl.BlockSpec((1, D, D), lambda r, i: (r, 0, 0)),
            pl.BlockSpec((1, DE, D), lambda r, i: (r, 0, 0)),
            pl.BlockSpec((1, 1, D), lambda r, i: (r, 0, 0)),
        ],
        out_specs=pl.BlockSpec((NP, 1, NT, CW), lambda r, i: (0, r, i, 0)),
        out_shape=jax.ShapeDtypeStruct((NP, R, E_PAD, CW), jnp.float32),
    )(xs, ea, qd, wk1, wk2, bk, wv1, wv2, bv)


def _agg_body(p_ref, wm_ref, bm_ref, o_ref):
    b = p_ref[:, 0]
    num = jnp.zeros((NT, D), jnp.float32)
    for p in range(NP - 1):
        jrow = lax.broadcasted_iota(jnp.int32, (CW, D), 0)
        dcol = lax.broadcasted_iota(jnp.int32, (CW, D), 1)
        selp = (jrow + p * CW == dcol).astype(jnp.float32)
        num = num + jnp.dot(b[p], selp, preferred_element_type=jnp.float32)
    jrow = lax.broadcasted_iota(jnp.int32, (CW, D), 0)
    hcol = lax.broadcasted_iota(jnp.int32, (CW, D), 1) // DH
    m = (jrow == hcol).astype(jnp.float32)
    den = jnp.dot(b[NP - 1], m, preferred_element_type=jnp.float32)
    agg = num / (den + 1e-16)
    o_ref[0] = _gelu(
        jnp.dot(agg, wm_ref[0], preferred_element_type=jnp.float32)
        + bm_ref[0]
    )


def _tc_agg(partials, wm, bm):
    return pl.pallas_call(
        _agg_body,
        grid=(R, NBLK_N),
        in_specs=[
            pl.BlockSpec((NP, 1, NT, CW), lambda r, i: (0, r, i, 0)),
            pl.BlockSpec((1, D, D), lambda r, i: (r, 0, 0)),
            pl.BlockSpec((1, 1, D), lambda r, i: (r, 0, 0)),
        ],
        out_specs=pl.BlockSpec((1, NT, D), lambda r, i: (r, i, 0)),
        out_shape=jax.ShapeDtypeStruct((R, N_PAD, D), jnp.float32),
    )(partials, wm, bm)


def _final_body(x_ref, ro_ref, iw1_ref, ib1_ref, iw2_ref, ib2_ref,
                mpaw1_ref, mpab1_ref, mpaw2_ref, mptw_ref, mptb_ref,
                mplg_ref, mplb_ref, cw1_ref, cw2_ref, cb_ref,
                lng_ref, lnb_ref, o_ref):
    x = x_ref[...]
    h1 = ib1_ref[...]
    for r in range(R):
        h1 = h1 + jnp.dot(ro_ref[r], iw1_ref[0, r * D:(r + 1) * D],
                          preferred_element_type=jnp.float32)
    h1 = _gelu(h1)
    logits = jnp.dot(h1, iw2_ref[...], preferred_element_type=jnp.float32) \
        + ib2_ref[...]
    logits = logits - jnp.max(logits, axis=-1, keepdims=True)
    ew = jnp.exp(logits)
    iw = ew / jnp.sum(ew, axis=-1, keepdims=True)
    inter = jnp.zeros((NT, D), jnp.float32)
    for r in range(R):
        inter = inter + iw[:, r:r + 1] * ro_ref[r]
    groups = ((2, 3), (4, 0), (1, 5))
    mps = []
    scs = []
    for gi in range(3):
        a, b = groups[gi]
        g = ro_ref[a] + ro_ref[b]
        mp = jnp.dot(g, mptw_ref[gi], preferred_element_type=jnp.float32) \
            + mptb_ref[gi]
        mps.append(mp)
        t = jnp.tanh(
            jnp.dot(mp, mpaw1_ref[...], preferred_element_type=jnp.float32)
            + mpab1_ref[...]
        )
        scs.append(jnp.sum(t * mpaw2_ref[...], axis=-1, keepdims=True))
    smax = jnp.maximum(jnp.maximum(scs[0], scs[1]), scs[2])
    e0 = jnp.exp(scs[0] - smax)
    e1 = jnp.exp(scs[1] - smax)
    e2 = jnp.exp(scs[2] - smax)
    esum = e0 + e1 + e2
    meta = (e0 * mps[0] + e1 * mps[1] + e2 * mps[2]) / esum
    meta = _layernorm(meta, mplg_ref[...], mplb_ref[...])
    comb = _gelu(
        jnp.dot(inter, cw1_ref[...], preferred_element_type=jnp.float32)
        + jnp.dot(meta, cw2_ref[...], preferred_element_type=jnp.float32)
        + cb_ref[...]
    )
    o_ref[...] = _layernorm(x + comb, lng_ref[...], lnb_ref[...])


def _tc_final(x_pad, ro, iw1, ib1, iw2, ib2, mpaw1, mpab1, mpaw2,
              mptw, mptb, mplg, mplb, cw1, cw2, cb, lng, lnb):
    full = lambda *s: pl.BlockSpec(s, lambda i: tuple(0 for _ in s))
    return pl.pallas_call(
        _final_body,
        grid=(NBLK_N,),
        in_specs=[
            pl.BlockSpec((NT, D), lambda i: (i, 0)),
            pl.BlockSpec((R, NT, D), lambda i: (0, i, 0)),
            pl.BlockSpec((1, R * D, D), lambda i: (0, 0, 0)),
            full(1, D),
            full(D, R),
            full(1, R),
            full(D, D // 2),
            full(1, D // 2),
            full(1, D // 2),
            full(3, D, D),
            pl.BlockSpec((3, 1, D), lambda i: (0, 0, 0)),
            full(1, D),
            full(1, D),
            full(D, D),
            full(D, D),
            full(1, D),
            full(1, D),
            full(1, D),
        ],
        out_specs=pl.BlockSpec((NT, D), lambda i: (i, 0)),
        out_shape=jax.ShapeDtypeStruct((N_PAD, D), jnp.float32),
    )(x_pad, ro, iw1, ib1, iw2, ib2, mpaw1, mpab1, mpaw2,
      mptw, mptb, mplg, mplb, cw1, cw2, cb, lng, lnb)


def _make_sc_gather(table_rows):
    mesh = plsc.VectorSubcoreMesh(core_axis_name="c", subcore_axis_name="s", num_cores=NC, num_subcores=NS)

    @functools.partial(
        pl.kernel,
        out_type=jax.ShapeDtypeStruct((GB, D), jnp.float32),
        mesh=mesh,
        scratch_types=[
            pltpu.VMEM((CS,), jnp.int32),
            pltpu.VMEM((CS, D), jnp.float32),
            pltpu.SemaphoreType.DMA,
        ],
    )
    def gk(table_hbm, idx_hbm, out_hbm, idx_v, rows_v, sem):
        wid = lax.axis_index("s") * NC + lax.axis_index("c")
        base = wid * PER_W

        def body(ci, _):
            off = base + ci * CS
            pltpu.sync_copy(idx_hbm.at[pl.ds(off, CS)], idx_v)
            pltpu.async_copy(table_hbm.at[idx_v], rows_v, sem).wait()
            pltpu.sync_copy(rows_v, out_hbm.at[pl.ds(off, CS)])
            return _

        lax.fori_loop(0, GCH, body, 0, unroll=False)

    return gk


def _sc_scatter(evx, dst_pad):
    mesh = plsc.VectorSubcoreMesh(core_axis_name="c", subcore_axis_name="s", num_cores=NC, num_subcores=NS)

    @functools.partial(
        pl.kernel,
        out_type=jax.ShapeDtypeStruct((NP, R, N_PAD, CW), jnp.float32),
        mesh=mesh,
        scratch_types=[
            pltpu.VMEM((CS,), jnp.int32),
            pltpu.VMEM((CS, CW), jnp.float32),
            pltpu.VMEM((RPS // 16, CW), jnp.float32),
            pltpu.VMEM_SHARED((N_PAD, CW), jnp.float32),
        ],
    )
    def sk(evx_hbm, dst_hbm, out_hbm, idx_v, data_v, zero_v, acc):
        cid = lax.axis_index("c")
        sid = lax.axis_index("s")
        zeros16 = jnp.zeros((16,), jnp.float32)

        def zb(i, _):
            zero_v[i, pl.ds(0, 16)] = zeros16
            return _

        lax.fori_loop(0, RPS // 16, zb, 0, unroll=False)
        row0 = sid * RPS
        ebase = sid * EPT

        def rel_body(r, _):
            for p in range(NP):
                def zi_body(zi, c):
                    pltpu.sync_copy(
                        zero_v,
                        acc.at[pl.ds(row0 + zi * (RPS // 16), RPS // 16)])
                    return c

                lax.fori_loop(0, 16, zi_body, 0, unroll=False)

                plsc.subcore_barrier()

                def ch(ci, c):
                    off = ebase + ci * CS
                    pltpu.sync_copy(dst_hbm.at[r, pl.ds(off, CS)], idx_v)
                    pltpu.sync_copy(
                        evx_hbm.at[p, r, pl.ds(off, CS)], data_v)
                    pltpu.sync_copy(data_v, acc.at[idx_v], add=True)
                    return c

                lax.fori_loop(0, SCH, ch, 0, unroll=False)

                plsc.subcore_barrier()

                pltpu.sync_copy(
                    acc.at[pl.ds(row0, RPS)],
                    out_hbm.at[p, r, pl.ds(row0, RPS)],
                )
            return _

        lax.fori_loop(0, R, rel_body, 0, unroll=False)

    return sk(evx, dst_pad)


def _sc_dbg(dst_pad):
    mesh = plsc.VectorSubcoreMesh(core_axis_name="c", subcore_axis_name="s",
                                  num_cores=NC, num_subcores=NS)

    @functools.partial(
        pl.kernel,
        out_type=jax.ShapeDtypeStruct((NC * NS * CS, CW), jnp.float32),
        mesh=mesh,
        scratch_types=[
            pltpu.VMEM((CS, CW), jnp.float32),
            pltpu.VMEM_SHARED((NC * NS * CS, CW), jnp.float32),
        ],
    )
    def dk(dst_hbm, out_hbm, data_v, acc):
        cid = lax.axis_index("c")
        sid = lax.axis_index("s")
        wid = sid * NC + cid
        zeros16 = jnp.zeros((16,), jnp.float32)

        def zb(i, c):
            data_v[i, pl.ds(0, 16)] = zeros16
            return c

        lax.fori_loop(0, CS, zb, 0, unroll=False)
        row0 = wid * CS
        pltpu.sync_copy(data_v, acc.at[pl.ds(row0, CS)])
        plsc.subcore_barrier()
        pltpu.sync_copy(acc.at[pl.ds(row0, CS)], out_hbm.at[pl.ds(row0, CS)])

    return dk(dst_pad)


@jax.jit
def kernel(x, edge_index, edge_attr, Wq, bq, Wk, bk, Wv, bv, prior, Wm, bm,
           iW1, ib1, iW2, ib2, mpaW1, mpab1, mpaW2, mptW, mptb, mplg, mplb,
           cW, cb, lng, lnb):
    f32 = jnp.float32
    x_pad = jnp.pad(x, ((0, N_PAD - N), (0, 0)))
    scale = jnp.repeat(prior, DH, axis=1) / np.sqrt(DH)
    wq_s = Wq * scale[:, None, :]
    bq_s = bq * scale

    src = edge_index[:, 0, :]
    dst = edge_index[:, 1, :]
    src_pad = jnp.pad(src, ((0, 0), (0, E_PAD - E)))
    dst_pad = jnp.pad(dst, ((0, 0), (0, E_PAD - E)))
    ea_pad = jnp.pad(edge_attr, ((0, 0), (0, E_PAD - E), (0, 0)))

    q_all = _tc_qall(x_pad, wq_s, bq_s.reshape(R, 1, D))

    gather_x = _make_sc_gather(N_PAD)
    gather_q = _make_sc_gather(R * N_PAD)
    xs = gather_x(x_pad, src_pad.reshape(-1))
    qidx = (dst_pad + (jnp.arange(R, dtype=jnp.int32) * N_PAD)[:, None])
    qd = gather_q(q_all.reshape(R * N_PAD, D), qidx.reshape(-1))

    wk1, wk2 = Wk[:, :D, :], Wk[:, D:, :]
    wv1, wv2 = Wv[:, :D, :], Wv[:, D:, :]
    evx = _tc_edges(
        xs.reshape(R, E_PAD, D), ea_pad, qd.reshape(R, E_PAD, D),
        wk1, wk2, bk.reshape(R, 1, D), wv1, wv2, bv.reshape(R, 1, D),
    )

    partials = jax.vmap(lambda evr, dr: jax.vmap(
        lambda ev1: jax.ops.segment_sum(ev1, dr, num_segments=N_PAD)
    )(evr), in_axes=(1, 0), out_axes=1)(evx, dst_pad)

    ro = _tc_agg(partials, Wm, bm.reshape(R, 1, D))

    out_pad = _tc_final(
        x_pad, ro,
        iW1.reshape(1, R * D, D), ib1.reshape(1, D), iW2, ib2.reshape(1, R),
        mpaW1, mpab1.reshape(1, D // 2), mpaW2.reshape(1, D // 2),
        mptW, mptb.reshape(3, 1, D), mplg.reshape(1, D), mplb.reshape(1, D),
        cW[:D, :], cW[D:, :], cb.reshape(1, D),
        lng.reshape(1, D), lnb.reshape(1, D),
    )
    return out_pad[:N].astype(f32)


# speedup vs baseline: 9.8947x; 9.8947x over previous
import functools

import jax
import jax.numpy as jnp
import numpy as np
from jax import lax
from jax.experimental import pallas as pl
from jax.experimental.pallas import tpu as pltpu
from jax.experimental.pallas import tpu_sc as plsc

N = 50000
E = 100000
R = 6
D = 128
DE = 16
H = 4
DH = D // H

NT = 256
N_PAD = 50176
E_PAD = 102400
NBLK_N = N_PAD // NT
NBLK_E = E_PAD // NT
NP = 9
CW = 16

NC, NS = 2, 16
CS = 128
GB = R * E_PAD
PER_W = GB // (NC * NS)
GCH = PER_W // CS
RPS = N_PAD // NS
EPT = E_PAD // NS
SCH = EPT // CS


def _gelu(z):
    return 0.5 * z * (1.0 + lax.erf(z * 0.7071067811865476))


def _layernorm(z, g, b, eps=1e-5):
    m = jnp.mean(z, axis=-1, keepdims=True)
    v = jnp.mean((z - m) ** 2, axis=-1, keepdims=True)
    return (z - m) / jnp.sqrt(v + eps) * g + b


def _proj_body(x_ref, w_ref, b_ref, o_ref):
    o_ref[0] = (
        jnp.dot(x_ref[...], w_ref[0], preferred_element_type=jnp.float32)
        + b_ref[0]
    )


def _tc_qall(x_pad, wq, bq):
    return pl.pallas_call(
        _proj_body,
        grid=(R, NBLK_N),
        in_specs=[
            pl.BlockSpec((NT, D), lambda r, i: (i, 0)),
            pl.BlockSpec((1, D, D), lambda r, i: (r, 0, 0)),
            pl.BlockSpec((1, 1, D), lambda r, i: (r, 0, 0)),
        ],
        out_specs=pl.BlockSpec((1, NT, D), lambda r, i: (r, i, 0)),
        out_shape=jax.ShapeDtypeStruct((R, N_PAD, D), jnp.float32),
    )(x_pad, wq, bq)


def _edge_body(xs_ref, ea_ref, qd_ref, wk1_ref, wk2_ref, bk_ref,
               wv1_ref, wv2_ref, bv_ref, o_ref):
    i = pl.program_id(1)
    xs = xs_ref[0]
    ea = ea_ref[0]
    qd = qd_ref[0]
    k = (jnp.dot(xs, wk1_ref[0], preferred_element_type=jnp.float32)
         + jnp.dot(ea, wk2_ref[0], preferred_element_type=jnp.float32)
         + bk_ref[0])
    v = (jnp.dot(xs, wv1_ref[0], preferred_element_type=jnp.float32)
         + jnp.dot(ea, wv2_ref[0], preferred_element_type=jnp.float32)
         + bv_ref[0])
    drow = lax.broadcasted_iota(jnp.int32, (D, H), 0) // DH
    hcol = lax.broadcasted_iota(jnp.int32, (D, H), 1)
    sel = (drow == hcol).astype(jnp.float32)
    s = jnp.dot(qd * k, sel, preferred_element_type=jnp.float32)
    rowid = i * NT + lax.broadcasted_iota(jnp.int32, (NT, 1), 0)
    e = jnp.where(rowid < E, jnp.exp(s), 0.0)
    selt = (jnp.transpose(hcol) == jnp.transpose(drow)).astype(jnp.float32)
    ev = v * jnp.dot(e, selt, preferred_element_type=jnp.float32)
    prow = lax.broadcasted_iota(jnp.int32, (H, CW), 0)
    pcol = lax.broadcasted_iota(jnp.int32, (H, CW), 1)
    pad = (prow == pcol).astype(jnp.float32)
    e16 = jnp.dot(e, pad, preferred_element_type=jnp.float32)
    o_ref[0] = jnp.concatenate([ev, e16], axis=-1)


def _tc_edges(xs, ea, qd, wk1, wk2, bk, wv1, wv2, bv):
    return pl.pallas_call(
        _edge_body,
        grid=(R, NBLK_E),
        in_specs=[
            pl.BlockSpec((1, NT, D), lambda r, i: (r, i, 0)),
            pl.BlockSpec((1, NT, DE), lambda r, i: (r, i, 0)),
            pl.BlockSpec((1, NT, D), lambda r, i: (r, i, 0)),
            pl.BlockSpec((1, D, D), lambda r, i: (r, 0, 0)),
            pl.BlockSpec((1, DE, D), lambda r, i: (r, 0, 0)),
            pl.BlockSpec((1, 1, D), lambda r, i: (r, 0, 0)),
            pl.BlockSpec((1, D, D), lambda r, i: (r, 0, 0)),
            pl.BlockSpec((1, DE, D), lambda r, i: (r, 0, 0)),
            pl.BlockSpec((1, 1, D), lambda r, i: (r, 0, 0)),
        ],
        out_specs=pl.BlockSpec((1, NT, D + CW), lambda r, i: (r, i, 0)),
        out_shape=jax.ShapeDtypeStruct((R, E_PAD, D + CW), jnp.float32),
    )(xs, ea, qd, wk1, wk2, bk, wv1, wv2, bv)


def _agg_body(p_ref, wm_ref, bm_ref, o_ref):
    b = p_ref[0]
    num = b[:, :D]
    jrow = lax.broadcasted_iota(jnp.int32, (CW, D), 0)
    hcol = lax.broadcasted_iota(jnp.int32, (CW, D), 1) // DH
    m = (jrow == hcol).astype(jnp.float32)
    den = jnp.dot(b[:, D:], m, preferred_element_type=jnp.float32)
    agg = num / (den + 1e-16)
    o_ref[0] = _gelu(
        jnp.dot(agg, wm_ref[0], preferred_element_type=jnp.float32)
        + bm_ref[0]
    )


def _tc_agg(partials, wm, bm):
    return pl.pallas_call(
        _agg_body,
        grid=(R, NBLK_N),
        in_specs=[
            pl.BlockSpec((1, NT, D + CW), lambda r, i: (r, i, 0)),
            pl.BlockSpec((1, D, D), lambda r, i: (r, 0, 0)),
            pl.BlockSpec((1, 1, D), lambda r, i: (r, 0, 0)),
        ],
        out_specs=pl.BlockSpec((1, NT, D), lambda r, i: (r, i, 0)),
        out_shape=jax.ShapeDtypeStruct((R, N_PAD, D), jnp.float32),
    )(partials, wm, bm)


def _final_body(x_ref, ro_ref, iw1_ref, ib1_ref, iw2_ref, ib2_ref,
                mpaw1_ref, mpab1_ref, mpaw2_ref, mptw_ref, mptb_ref,
                mplg_ref, mplb_ref, cw1_ref, cw2_ref, cb_ref,
                lng_ref, lnb_ref, o_ref):
    x = x_ref[...]
    h1 = ib1_ref[...]
    for r in range(R):
        h1 = h1 + jnp.dot(ro_ref[r], iw1_ref[0, r * D:(r + 1) * D],
                          preferred_element_type=jnp.float32)
    h1 = _gelu(h1)
    logits = jnp.dot(h1, iw2_ref[...], preferred_element_type=jnp.float32) \
        + ib2_ref[...]
    logits = logits - jnp.max(logits, axis=-1, keepdims=True)
    ew = jnp.exp(logits)
    iw = ew / jnp.sum(ew, axis=-1, keepdims=True)
    inter = jnp.zeros((NT, D), jnp.float32)
    for r in range(R):
        inter = inter + iw[:, r:r + 1] * ro_ref[r]
    groups = ((2, 3), (4, 0), (1, 5))
    mps = []
    scs = []
    for gi in range(3):
        a, b = groups[gi]
        g = ro_ref[a] + ro_ref[b]
        mp = jnp.dot(g, mptw_ref[gi], preferred_element_type=jnp.float32) \
            + mptb_ref[gi]
        mps.append(mp)
        t = jnp.tanh(
            jnp.dot(mp, mpaw1_ref[...], preferred_element_type=jnp.float32)
            + mpab1_ref[...]
        )
        scs.append(jnp.sum(t * mpaw2_ref[...], axis=-1, keepdims=True))
    smax = jnp.maximum(jnp.maximum(scs[0], scs[1]), scs[2])
    e0 = jnp.exp(scs[0] - smax)
    e1 = jnp.exp(scs[1] - smax)
    e2 = jnp.exp(scs[2] - smax)
    esum = e0 + e1 + e2
    meta = (e0 * mps[0] + e1 * mps[1] + e2 * mps[2]) / esum
    meta = _layernorm(meta, mplg_ref[...], mplb_ref[...])
    comb = _gelu(
        jnp.dot(inter, cw1_ref[...], preferred_element_type=jnp.float32)
        + jnp.dot(meta, cw2_ref[...], preferred_element_type=jnp.float32)
        + cb_ref[...]
    )
    o_ref[...] = _layernorm(x + comb, lng_ref[...], lnb_ref[...])


def _tc_final(x_pad, ro, iw1, ib1, iw2, ib2, mpaw1, mpab1, mpaw2,
              mptw, mptb, mplg, mplb, cw1, cw2, cb, lng, lnb):
    full = lambda *s: pl.BlockSpec(s, lambda i: tuple(0 for _ in s))
    return pl.pallas_call(
        _final_body,
        grid=(NBLK_N,),
        in_specs=[
            pl.BlockSpec((NT, D), lambda i: (i, 0)),
            pl.BlockSpec((R, NT, D), lambda i: (0, i, 0)),
            pl.BlockSpec((1, R * D, D), lambda i: (0, 0, 0)),
            full(1, D),
            full(D, R),
            full(1, R),
            full(D, D // 2),
            full(1, D // 2),
            full(1, D // 2),
            full(3, D, D),
            pl.BlockSpec((3, 1, D), lambda i: (0, 0, 0)),
            full(1, D),
            full(1, D),
            full(D, D),
            full(D, D),
            full(1, D),
            full(1, D),
            full(1, D),
        ],
        out_specs=pl.BlockSpec((NT, D), lambda i: (i, 0)),
        out_shape=jax.ShapeDtypeStruct((N_PAD, D), jnp.float32),
    )(x_pad, ro, iw1, ib1, iw2, ib2, mpaw1, mpab1, mpaw2,
      mptw, mptb, mplg, mplb, cw1, cw2, cb, lng, lnb)


def _make_sc_gather(table_rows):
    mesh = plsc.VectorSubcoreMesh(core_axis_name="c", subcore_axis_name="s", num_cores=NC, num_subcores=NS)

    @functools.partial(
        pl.kernel,
        out_type=jax.ShapeDtypeStruct((GB, D), jnp.float32),
        mesh=mesh,
        scratch_types=[
            pltpu.VMEM((CS,), jnp.int32),
            pltpu.VMEM((CS, D), jnp.float32),
            pltpu.SemaphoreType.DMA,
        ],
    )
    def gk(table_hbm, idx_hbm, out_hbm, idx_v, rows_v, sem):
        wid = lax.axis_index("s") * NC + lax.axis_index("c")
        base = wid * PER_W

        def body(ci, _):
            off = base + ci * CS
            pltpu.sync_copy(idx_hbm.at[pl.ds(off, CS)], idx_v)
            pltpu.async_copy(table_hbm.at[idx_v], rows_v, sem).wait()
            pltpu.sync_copy(rows_v, out_hbm.at[pl.ds(off, CS)])
            return _

        lax.fori_loop(0, GCH, body, 0, unroll=False)

    return gk


def _sc_scatter(evx, dst_pad):
    mesh = plsc.VectorSubcoreMesh(core_axis_name="c", subcore_axis_name="s", num_cores=NC, num_subcores=NS)

    @functools.partial(
        pl.kernel,
        out_type=jax.ShapeDtypeStruct((NP, R, N_PAD, CW), jnp.float32),
        mesh=mesh,
        scratch_types=[
            pltpu.VMEM((CS,), jnp.int32),
            pltpu.VMEM((CS, CW), jnp.float32),
            pltpu.VMEM((RPS // 16, CW), jnp.float32),
            pltpu.VMEM_SHARED((N_PAD, CW), jnp.float32),
        ],
    )
    def sk(evx_hbm, dst_hbm, out_hbm, idx_v, data_v, zero_v, acc):
        cid = lax.axis_index("c")
        sid = lax.axis_index("s")
        zeros16 = jnp.zeros((16,), jnp.float32)

        def zb(i, _):
            zero_v[i, pl.ds(0, 16)] = zeros16
            return _

        lax.fori_loop(0, RPS // 16, zb, 0, unroll=False)
        row0 = sid * RPS
        ebase = sid * EPT

        def rel_body(r, _):
            for p in range(NP):
                def zi_body(zi, c):
                    pltpu.sync_copy(
                        zero_v,
                        acc.at[pl.ds(row0 + zi * (RPS // 16), RPS // 16)])
                    return c

                lax.fori_loop(0, 16, zi_body, 0, unroll=False)

                plsc.subcore_barrier()

                def ch(ci, c):
                    off = ebase + ci * CS
                    pltpu.sync_copy(dst_hbm.at[r, pl.ds(off, CS)], idx_v)
                    pltpu.sync_copy(
                        evx_hbm.at[p, r, pl.ds(off, CS)], data_v)
                    pltpu.sync_copy(data_v, acc.at[idx_v], add=True)
                    return c

                lax.fori_loop(0, SCH, ch, 0, unroll=False)

                plsc.subcore_barrier()

                pltpu.sync_copy(
                    acc.at[pl.ds(row0, RPS)],
                    out_hbm.at[p, r, pl.ds(row0, RPS)],
                )
            return _

        lax.fori_loop(0, R, rel_body, 0, unroll=False)

    return sk(evx, dst_pad)


def _sc_dbg(dst_pad):
    mesh = plsc.VectorSubcoreMesh(core_axis_name="c", subcore_axis_name="s",
                                  num_cores=NC, num_subcores=NS)

    @functools.partial(
        pl.kernel,
        out_type=jax.ShapeDtypeStruct((NC * NS * CS, CW), jnp.float32),
        mesh=mesh,
        scratch_types=[
            pltpu.VMEM((CS, CW), jnp.float32),
            pltpu.VMEM_SHARED((NC * NS * CS, CW), jnp.float32),
        ],
    )
    def dk(dst_hbm, out_hbm, data_v, acc):
        cid = lax.axis_index("c")
        sid = lax.axis_index("s")
        wid = sid * NC + cid
        zeros16 = jnp.zeros((16,), jnp.float32)

        def zb(i, c):
            data_v[i, pl.ds(0, 16)] = zeros16
            return c

        lax.fori_loop(0, CS, zb, 0, unroll=False)
        row0 = wid * CS
        pltpu.sync_copy(data_v, acc.at[pl.ds(row0, CS)])
        plsc.subcore_barrier()
        pltpu.sync_copy(acc.at[pl.ds(row0, CS)], out_hbm.at[pl.ds(row0, CS)])

    return dk(dst_pad)


@jax.jit
def kernel(x, edge_index, edge_attr, Wq, bq, Wk, bk, Wv, bv, prior, Wm, bm,
           iW1, ib1, iW2, ib2, mpaW1, mpab1, mpaW2, mptW, mptb, mplg, mplb,
           cW, cb, lng, lnb):
    f32 = jnp.float32
    x_pad = jnp.pad(x, ((0, N_PAD - N), (0, 0)))
    scale = jnp.repeat(prior, DH, axis=1) / np.sqrt(DH)
    wq_s = Wq * scale[:, None, :]
    bq_s = bq * scale

    src = edge_index[:, 0, :]
    dst = edge_index[:, 1, :]
    src_pad = jnp.pad(src, ((0, 0), (0, E_PAD - E)))
    dst_pad = jnp.pad(dst, ((0, 0), (0, E_PAD - E)))
    ea_pad = jnp.pad(edge_attr, ((0, 0), (0, E_PAD - E), (0, 0)))

    q_all = _tc_qall(x_pad, wq_s, bq_s.reshape(R, 1, D))

    gather_x = _make_sc_gather(N_PAD)
    gather_q = _make_sc_gather(R * N_PAD)
    xs = gather_x(x_pad, src_pad.reshape(-1))
    qidx = (dst_pad + (jnp.arange(R, dtype=jnp.int32) * N_PAD)[:, None])
    qd = gather_q(q_all.reshape(R * N_PAD, D), qidx.reshape(-1))

    wk1, wk2 = Wk[:, :D, :], Wk[:, D:, :]
    wv1, wv2 = Wv[:, :D, :], Wv[:, D:, :]
    evx = _tc_edges(
        xs.reshape(R, E_PAD, D), ea_pad, qd.reshape(R, E_PAD, D),
        wk1, wk2, bk.reshape(R, 1, D), wv1, wv2, bv.reshape(R, 1, D),
    )

    partials = jax.vmap(
        lambda evr, dr: jax.ops.segment_sum(evr, dr, num_segments=N_PAD)
    )(evx, dst_pad)

    ro = _tc_agg(partials, Wm, bm.reshape(R, 1, D))

    out_pad = _tc_final(
        x_pad, ro,
        iW1.reshape(1, R * D, D), ib1.reshape(1, D), iW2, ib2.reshape(1, R),
        mpaW1, mpab1.reshape(1, D // 2), mpaW2.reshape(1, D // 2),
        mptW, mptb.reshape(3, 1, D), mplg.reshape(1, D), mplb.reshape(1, D),
        cW[:D, :], cW[D:, :], cb.reshape(1, D),
        lng.reshape(1, D), lnb.reshape(1, D),
    )
    return out_pad[:N].astype(f32)


# speedup vs baseline: 10.0953x; 1.0203x over previous
import functools

import jax
import jax.numpy as jnp
import numpy as np
from jax import lax
from jax.experimental import pallas as pl
from jax.experimental.pallas import tpu as pltpu
from jax.experimental.pallas import tpu_sc as plsc

N = 50000
E = 100000
R = 6
D = 128
DE = 16
H = 4
DH = D // H

NT = 256
N_PAD = 50176
E_PAD = 102400
NBLK_N = N_PAD // NT
NBLK_E = E_PAD // NT
NP = 9
CW = 16

NC, NS = 2, 16
CS = 128
GB = R * E_PAD
PER_W = GB // (NC * NS)
GCH = PER_W // CS
RPS = N_PAD // NS
EPT = E_PAD // NS
SCH = EPT // CS


def _gelu(z):
    return 0.5 * z * (1.0 + lax.erf(z * 0.7071067811865476))


def _layernorm(z, g, b, eps=1e-5):
    m = jnp.mean(z, axis=-1, keepdims=True)
    v = jnp.mean((z - m) ** 2, axis=-1, keepdims=True)
    return (z - m) / jnp.sqrt(v + eps) * g + b


def _proj_body(x_ref, w_ref, b_ref, o_ref):
    o_ref[0] = (
        jnp.dot(x_ref[...], w_ref[0], preferred_element_type=jnp.float32)
        + b_ref[0]
    )


def _tc_qall(x_pad, wq, bq):
    return pl.pallas_call(
        _proj_body,
        grid=(R, NBLK_N),
        in_specs=[
            pl.BlockSpec((NT, D), lambda r, i: (i, 0)),
            pl.BlockSpec((1, D, D), lambda r, i: (r, 0, 0)),
            pl.BlockSpec((1, 1, D), lambda r, i: (r, 0, 0)),
        ],
        out_specs=pl.BlockSpec((1, NT, D), lambda r, i: (r, i, 0)),
        out_shape=jax.ShapeDtypeStruct((R, N_PAD, D), jnp.float32),
    )(x_pad, wq, bq)


def _edge_body(xs_ref, ea_ref, qd_ref, wk1_ref, wk2_ref, bk_ref,
               wv1_ref, wv2_ref, bv_ref, o_ref):
    i = pl.program_id(1)
    xs = xs_ref[0]
    ea = ea_ref[0]
    qd = qd_ref[0]
    k = (jnp.dot(xs, wk1_ref[0], preferred_element_type=jnp.float32)
         + jnp.dot(ea, wk2_ref[0], preferred_element_type=jnp.float32)
         + bk_ref[0])
    v = (jnp.dot(xs, wv1_ref[0], preferred_element_type=jnp.float32)
         + jnp.dot(ea, wv2_ref[0], preferred_element_type=jnp.float32)
         + bv_ref[0])
    drow = lax.broadcasted_iota(jnp.int32, (D, H), 0) // DH
    hcol = lax.broadcasted_iota(jnp.int32, (D, H), 1)
    sel = (drow == hcol).astype(jnp.float32)
    s = jnp.dot(qd * k, sel, preferred_element_type=jnp.float32)
    rowid = i * NT + lax.broadcasted_iota(jnp.int32, (NT, 1), 0)
    e = jnp.where(rowid < E, jnp.exp(s), 0.0)
    selt = (jnp.transpose(hcol) == jnp.transpose(drow)).astype(jnp.float32)
    ev = v * jnp.dot(e, selt, preferred_element_type=jnp.float32)
    prow = lax.broadcasted_iota(jnp.int32, (H, CW), 0)
    pcol = lax.broadcasted_iota(jnp.int32, (H, CW), 1)
    pad = (prow == pcol).astype(jnp.float32)
    e16 = jnp.dot(e, pad, preferred_element_type=jnp.float32)
    o_ref[0] = jnp.concatenate([ev, e16], axis=-1)


def _tc_edges(xs, ea, qd, wk1, wk2, bk, wv1, wv2, bv):
    return pl.pallas_call(
        _edge_body,
        grid=(R, NBLK_E),
        in_specs=[
            pl.BlockSpec((1, NT, D), lambda r, i: (r, i, 0)),
            pl.BlockSpec((1, NT, DE), lambda r, i: (r, i, 0)),
            pl.BlockSpec((1, NT, D), lambda r, i: (r, i, 0)),
            pl.BlockSpec((1, D, D), lambda r, i: (r, 0, 0)),
            pl.BlockSpec((1, DE, D), lambda r, i: (r, 0, 0)),
            pl.BlockSpec((1, 1, D), lambda r, i: (r, 0, 0)),
            pl.BlockSpec((1, D, D), lambda r, i: (r, 0, 0)),
            pl.BlockSpec((1, DE, D), lambda r, i: (r, 0, 0)),
            pl.BlockSpec((1, 1, D), lambda r, i: (r, 0, 0)),
        ],
        out_specs=pl.BlockSpec((1, NT, D + CW), lambda r, i: (r, i, 0)),
        out_shape=jax.ShapeDtypeStruct((R, E_PAD, D + CW), jnp.float32),
    )(xs, ea, qd, wk1, wk2, bk, wv1, wv2, bv)


def _agg_body(p_ref, wm_ref, bm_ref, o_ref):
    b = p_ref[0]
    num = b[:, :D]
    jrow = lax.broadcasted_iota(jnp.int32, (CW, D), 0)
    hcol = lax.broadcasted_iota(jnp.int32, (CW, D), 1) // DH
    m = (jrow == hcol).astype(jnp.float32)
    den = jnp.dot(b[:, D:], m, preferred_element_type=jnp.float32)
    agg = num / (den + 1e-16)
    o_ref[0] = _gelu(
        jnp.dot(agg, wm_ref[0], preferred_element_type=jnp.float32)
        + bm_ref[0]
    )


def _tc_agg(partials, wm, bm):
    return pl.pallas_call(
        _agg_body,
        grid=(R, NBLK_N),
        in_specs=[
            pl.BlockSpec((1, NT, D + CW), lambda r, i: (r, i, 0)),
            pl.BlockSpec((1, D, D), lambda r, i: (r, 0, 0)),
            pl.BlockSpec((1, 1, D), lambda r, i: (r, 0, 0)),
        ],
        out_specs=pl.BlockSpec((1, NT, D), lambda r, i: (r, i, 0)),
        out_shape=jax.ShapeDtypeStruct((R, N_PAD, D), jnp.float32),
    )(partials, wm, bm)


def _final_body(x_ref, ro_ref, iw1_ref, ib1_ref, iw2_ref, ib2_ref,
                mpaw1_ref, mpab1_ref, mpaw2_ref, mptw_ref, mptb_ref,
                mplg_ref, mplb_ref, cw1_ref, cw2_ref, cb_ref,
                lng_ref, lnb_ref, o_ref):
    x = x_ref[...]
    h1 = ib1_ref[...]
    for r in range(R):
        h1 = h1 + jnp.dot(ro_ref[r], iw1_ref[0, r * D:(r + 1) * D],
                          preferred_element_type=jnp.float32)
    h1 = _gelu(h1)
    logits = jnp.dot(h1, iw2_ref[...], preferred_element_type=jnp.float32) \
        + ib2_ref[...]
    logits = logits - jnp.max(logits, axis=-1, keepdims=True)
    ew = jnp.exp(logits)
    iw = ew / jnp.sum(ew, axis=-1, keepdims=True)
    inter = jnp.zeros((NT, D), jnp.float32)
    for r in range(R):
        inter = inter + iw[:, r:r + 1] * ro_ref[r]
    groups = ((2, 3), (4, 0), (1, 5))
    mps = []
    scs = []
    for gi in range(3):
        a, b = groups[gi]
        g = ro_ref[a] + ro_ref[b]
        mp = jnp.dot(g, mptw_ref[gi], preferred_element_type=jnp.float32) \
            + mptb_ref[gi]
        mps.append(mp)
        t = jnp.tanh(
            jnp.dot(mp, mpaw1_ref[...], preferred_element_type=jnp.float32)
            + mpab1_ref[...]
        )
        scs.append(jnp.sum(t * mpaw2_ref[...], axis=-1, keepdims=True))
    smax = jnp.maximum(jnp.maximum(scs[0], scs[1]), scs[2])
    e0 = jnp.exp(scs[0] - smax)
    e1 = jnp.exp(scs[1] - smax)
    e2 = jnp.exp(scs[2] - smax)
    esum = e0 + e1 + e2
    meta = (e0 * mps[0] + e1 * mps[1] + e2 * mps[2]) / esum
    meta = _layernorm(meta, mplg_ref[...], mplb_ref[...])
    comb = _gelu(
        jnp.dot(inter, cw1_ref[...], preferred_element_type=jnp.float32)
        + jnp.dot(meta, cw2_ref[...], preferred_element_type=jnp.float32)
        + cb_ref[...]
    )
    o_ref[...] = _layernorm(x + comb, lng_ref[...], lnb_ref[...])


def _tc_final(x_pad, ro, iw1, ib1, iw2, ib2, mpaw1, mpab1, mpaw2,
              mptw, mptb, mplg, mplb, cw1, cw2, cb, lng, lnb):
    full = lambda *s: pl.BlockSpec(s, lambda i: tuple(0 for _ in s))
    return pl.pallas_call(
        _final_body,
        grid=(NBLK_N,),
        in_specs=[
            pl.BlockSpec((NT, D), lambda i: (i, 0)),
            pl.BlockSpec((R, NT, D), lambda i: (0, i, 0)),
            pl.BlockSpec((1, R * D, D), lambda i: (0, 0, 0)),
            full(1, D),
            full(D, R),
            full(1, R),
            full(D, D // 2),
            full(1, D // 2),
            full(1, D // 2),
            full(3, D, D),
            pl.BlockSpec((3, 1, D), lambda i: (0, 0, 0)),
            full(1, D),
            full(1, D),
            full(D, D),
            full(D, D),
            full(1, D),
            full(1, D),
            full(1, D),
        ],
        out_specs=pl.BlockSpec((NT, D), lambda i: (i, 0)),
        out_shape=jax.ShapeDtypeStruct((N_PAD, D), jnp.float32),
    )(x_pad, ro, iw1, ib1, iw2, ib2, mpaw1, mpab1, mpaw2,
      mptw, mptb, mplg, mplb, cw1, cw2, cb, lng, lnb)


def _make_sc_gather(table_rows):
    mesh = plsc.VectorSubcoreMesh(core_axis_name="c", subcore_axis_name="s", num_cores=NC, num_subcores=NS)

    @functools.partial(
        pl.kernel,
        out_type=jax.ShapeDtypeStruct((GB, D), jnp.float32),
        mesh=mesh,
        scratch_types=[
            pltpu.VMEM((PER_W,), jnp.int32),
            pltpu.VMEM((CS, D), jnp.float32),
            pltpu.VMEM((CS, D), jnp.float32),
            pltpu.SemaphoreType.DMA,
            pltpu.SemaphoreType.DMA,
        ],
    )
    def gk(table_hbm, idx_hbm, out_hbm, idx_v, rows_v0, rows_v1, s0, s1):
        wid = lax.axis_index("s") * NC + lax.axis_index("c")
        base = wid * PER_W
        pltpu.sync_copy(idx_hbm.at[pl.ds(base, PER_W)], idx_v)
        pltpu.async_copy(
            table_hbm.at[idx_v.at[pl.ds(0, CS)]], rows_v0, s0)

        def body(ci2, _):
            for b in range(2):
                ci = ci2 * 2 + b
                buf = rows_v0 if b == 0 else rows_v1
                sem = s0 if b == 0 else s1
                nbuf = rows_v1 if b == 0 else rows_v0
                nsem = s1 if b == 0 else s0
                pltpu.make_async_copy(
                    table_hbm.at[idx_v.at[pl.ds(0, CS)]], buf, sem).wait()

                @pl.when(ci + 1 < GCH)
                def _nxt():
                    pltpu.async_copy(
                        table_hbm.at[idx_v.at[pl.ds((ci + 1) * CS, CS)]],
                        nbuf, nsem)

                pltpu.sync_copy(buf, out_hbm.at[pl.ds(base + ci * CS, CS)])
            return _

        lax.fori_loop(0, GCH // 2, body, 0, unroll=False)

    return gk


def _sc_scatter(evx, dst_pad):
    mesh = plsc.VectorSubcoreMesh(core_axis_name="c", subcore_axis_name="s", num_cores=NC, num_subcores=NS)

    @functools.partial(
        pl.kernel,
        out_type=jax.ShapeDtypeStruct((NP, R, N_PAD, CW), jnp.float32),
        mesh=mesh,
        scratch_types=[
            pltpu.VMEM((CS,), jnp.int32),
            pltpu.VMEM((CS, CW), jnp.float32),
            pltpu.VMEM((RPS // 16, CW), jnp.float32),
            pltpu.VMEM_SHARED((N_PAD, CW), jnp.float32),
        ],
    )
    def sk(evx_hbm, dst_hbm, out_hbm, idx_v, data_v, zero_v, acc):
        cid = lax.axis_index("c")
        sid = lax.axis_index("s")
        zeros16 = jnp.zeros((16,), jnp.float32)

        def zb(i, _):
            zero_v[i, pl.ds(0, 16)] = zeros16
            return _

        lax.fori_loop(0, RPS // 16, zb, 0, unroll=False)
        row0 = sid * RPS
        ebase = sid * EPT

        def rel_body(r, _):
            for p in range(NP):
                def zi_body(zi, c):
                    pltpu.sync_copy(
                        zero_v,
                        acc.at[pl.ds(row0 + zi * (RPS // 16), RPS // 16)])
                    return c

                lax.fori_loop(0, 16, zi_body, 0, unroll=False)

                plsc.subcore_barrier()

                def ch(ci, c):
                    off = ebase + ci * CS
                    pltpu.sync_copy(dst_hbm.at[r, pl.ds(off, CS)], idx_v)
                    pltpu.sync_copy(
                        evx_hbm.at[p, r, pl.ds(off, CS)], data_v)
                    pltpu.sync_copy(data_v, acc.at[idx_v], add=True)
                    return c

                lax.fori_loop(0, SCH, ch, 0, unroll=False)

                plsc.subcore_barrier()

                pltpu.sync_copy(
                    acc.at[pl.ds(row0, RPS)],
                    out_hbm.at[p, r, pl.ds(row0, RPS)],
                )
            return _

        lax.fori_loop(0, R, rel_body, 0, unroll=False)

    return sk(evx, dst_pad)


def _sc_dbg(dst_pad):
    mesh = plsc.VectorSubcoreMesh(core_axis_name="c", subcore_axis_name="s",
                                  num_cores=NC, num_subcores=NS)

    @functools.partial(
        pl.kernel,
        out_type=jax.ShapeDtypeStruct((NC * NS * CS, CW), jnp.float32),
        mesh=mesh,
        scratch_types=[
            pltpu.VMEM((CS, CW), jnp.float32),
            pltpu.VMEM_SHARED((NC * NS * CS, CW), jnp.float32),
        ],
    )
    def dk(dst_hbm, out_hbm, data_v, acc):
        cid = lax.axis_index("c")
        sid = lax.axis_index("s")
        wid = sid * NC + cid
        zeros16 = jnp.zeros((16,), jnp.float32)

        def zb(i, c):
            data_v[i, pl.ds(0, 16)] = zeros16
            return c

        lax.fori_loop(0, CS, zb, 0, unroll=False)
        row0 = wid * CS
        pltpu.sync_copy(data_v, acc.at[pl.ds(row0, CS)])
        plsc.subcore_barrier()
        pltpu.sync_copy(acc.at[pl.ds(row0, CS)], out_hbm.at[pl.ds(row0, CS)])

    return dk(dst_pad)


@jax.jit
def kernel(x, edge_index, edge_attr, Wq, bq, Wk, bk, Wv, bv, prior, Wm, bm,
           iW1, ib1, iW2, ib2, mpaW1, mpab1, mpaW2, mptW, mptb, mplg, mplb,
           cW, cb, lng, lnb):
    f32 = jnp.float32
    x_pad = jnp.pad(x, ((0, N_PAD - N), (0, 0)))
    scale = jnp.repeat(prior, DH, axis=1) / np.sqrt(DH)
    wq_s = Wq * scale[:, None, :]
    bq_s = bq * scale

    src = edge_index[:, 0, :]
    dst = edge_index[:, 1, :]
    src_pad = jnp.pad(src, ((0, 0), (0, E_PAD - E)))
    dst_pad = jnp.pad(dst, ((0, 0), (0, E_PAD - E)))
    ea_pad = jnp.pad(edge_attr, ((0, 0), (0, E_PAD - E), (0, 0)))

    q_all = _tc_qall(x_pad, wq_s, bq_s.reshape(R, 1, D))

    gather_x = _make_sc_gather(N_PAD)
    gather_q = _make_sc_gather(R * N_PAD)
    xs = gather_x(x_pad, src_pad.reshape(-1))
    qidx = (dst_pad + (jnp.arange(R, dtype=jnp.int32) * N_PAD)[:, None])
    qd = gather_q(q_all.reshape(R * N_PAD, D), qidx.reshape(-1))

    wk1, wk2 = Wk[:, :D, :], Wk[:, D:, :]
    wv1, wv2 = Wv[:, :D, :], Wv[:, D:, :]
    evx = _tc_edges(
        xs.reshape(R, E_PAD, D), ea_pad, qd.reshape(R, E_PAD, D),
        wk1, wk2, bk.reshape(R, 1, D), wv1, wv2, bv.reshape(R, 1, D),
    )

    partials = jax.vmap(
        lambda evr, dr: jax.ops.segment_sum(evr, dr, num_segments=N_PAD)
    )(evx, dst_pad)

    ro = _tc_agg(partials, Wm, bm.reshape(R, 1, D))

    out_pad = _tc_final(
        x_pad, ro,
        iW1.reshape(1, R * D, D), ib1.reshape(1, D), iW2, ib2.reshape(1, R),
        mpaW1, mpab1.reshape(1, D // 2), mpaW2.reshape(1, D // 2),
        mptW, mptb.reshape(3, 1, D), mplg.reshape(1, D), mplb.reshape(1, D),
        cW[:D, :], cW[D:, :], cb.reshape(1, D),
        lng.reshape(1, D), lnb.reshape(1, D),
    )
    return out_pad[:N].astype(f32)
